# baseline (speedup 1.0000x reference)
import functools

import jax
import jax.numpy as jnp
from jax import lax
from jax.experimental import pallas as pl
from jax.experimental.pallas import tpu as pltpu
from jax.experimental.pallas import tpu_sc as plsc

S = 64
K = 256


def _spmv_body(x_hbm, idx_hbm, vals_hbm, out_hbm,
               x_s, idx_s, vals_s, acc_s, sem):
    cp_x = pltpu.make_async_copy(x_hbm, x_s, sem)
    cp_i = pltpu.make_async_copy(idx_hbm, idx_s, sem)
    cp_v = pltpu.make_async_copy(vals_hbm, vals_s, sem)
    cp_x.start()
    cp_i.start()
    cp_v.start()

    def zstep(j, carry):
        acc_s[j] = 0.0
        return carry

    lax.fori_loop(0, S, zstep, 0)

    cp_x.wait()
    cp_i.wait()
    cp_v.wait()

    def step(k, carry):
        r = idx_s[0, k]
        c = idx_s[1, k]
        acc_s[r] = acc_s[r] + vals_s[k] * x_s[c]
        return carry

    lax.fori_loop(0, K, step, 0)

    pltpu.sync_copy(acc_s, out_hbm)


@jax.jit
def _spmv(x, idx, vals):
    mesh = plsc.ScalarSubcoreMesh(axis_name="c", num_cores=1)
    return pl.kernel(
        _spmv_body,
        out_type=jax.ShapeDtypeStruct((S,), jnp.float32),
        mesh=mesh,
        scratch_types=[
            pltpu.SMEM((S,), jnp.float32),
            pltpu.SMEM((2, K), jnp.int32),
            pltpu.SMEM((K,), jnp.float32),
            pltpu.SMEM((S,), jnp.float32),
            pltpu.SemaphoreType.DMA,
        ],
        compiler_params=pltpu.CompilerParams(needs_layout_passes=False),
    )(x, idx, vals)


def kernel(x, indices, values):
    return _spmv(x, indices.astype(jnp.int32), values)


# speedup vs baseline: 1.0115x; 1.0115x over previous
import functools

import jax
import jax.numpy as jnp
from jax import lax
from jax.experimental import pallas as pl
from jax.experimental.pallas import tpu as pltpu
from jax.experimental.pallas import tpu_sc as plsc

S = 64
K = 256
L = 16


def _spmv_body(x_hbm, idx_hbm, vals_hbm, out_hbm,
               x_v, idx_v, vals_v, acc_a, acc_b, sem):
    cp_x = pltpu.make_async_copy(x_hbm, x_v, sem)
    cp_i = pltpu.make_async_copy(idx_hbm, idx_v, sem)
    cp_v = pltpu.make_async_copy(vals_hbm, vals_v, sem)
    cp_x.start()
    cp_i.start()
    cp_v.start()

    zero = jnp.zeros((L,), jnp.float32)
    for j in range(S // L):
        acc_a[pl.ds(j * L, L)] = zero
        acc_b[pl.ds(j * L, L)] = zero

    cp_x.wait()
    cp_i.wait()
    cp_v.wait()

    for i in range(K // L):
        r = idx_v[0, pl.ds(i * L, L)]
        c = idx_v[1, pl.ds(i * L, L)]
        v = vals_v[pl.ds(i * L, L)]
        g = plsc.load_gather(x_v, [c])
        acc = acc_a if i % 2 == 0 else acc_b
        plsc.addupdate_scatter(acc, [r], v * g)

    for j in range(S // L):
        sl = pl.ds(j * L, L)
        acc_a[sl] = acc_a[sl] + acc_b[sl]

    pltpu.sync_copy(acc_a, out_hbm)


@jax.jit
def _spmv(x, idx, vals):
    mesh = plsc.VectorSubcoreMesh(
        core_axis_name="c", subcore_axis_name="s",
        num_cores=1, num_subcores=1)
    return pl.kernel(
        _spmv_body,
        out_type=jax.ShapeDtypeStruct((S,), jnp.float32),
        mesh=mesh,
        scratch_types=[
            pltpu.VMEM((S,), jnp.float32),
            pltpu.VMEM((2, K), jnp.int32),
            pltpu.VMEM((K,), jnp.float32),
            pltpu.VMEM((S,), jnp.float32),
            pltpu.VMEM((S,), jnp.float32),
            pltpu.SemaphoreType.DMA,
        ],
        compiler_params=pltpu.CompilerParams(needs_layout_passes=False),
    )(x, idx, vals)


def kernel(x, indices, values):
    return _spmv(x, indices.astype(jnp.int32), values)
